# Initial kernel scaffold; baseline (speedup 1.0000x reference)
import jax
import jax.numpy as jnp
from jax.experimental import pallas as pl


def kernel(image_tokens, adaln_input, adaln_w, adaln_b, q_w, q_b, k_w, k_b, v_w, v_b, o_w, o_b, q_rms, k_rms, gate_w, experts_w1, experts_w2, experts_w3, shared_w1, shared_w2, shared_w3):
    raise NotImplementedError("write your pallas kernel here")


# speedup vs baseline: 1.3698x; 1.3698x over previous
import functools

import jax
import jax.numpy as jnp
from jax.experimental import pallas as pl
from jax.experimental.pallas import tpu as pltpu

F32 = jnp.float32
BF16 = jnp.bfloat16


def _silu(x):
    return x * jax.nn.sigmoid(x)


def _ln(x, eps=1e-6):
    mu = jnp.mean(x, axis=-1, keepdims=True)
    var = jnp.mean((x - mu) * (x - mu), axis=-1, keepdims=True)
    return (x - mu) * jax.lax.rsqrt(var + eps)


def _dot_t(a, b, out_dtype=F32):
    return jax.lax.dot_general(a, b, (((1,), (1,)), ((), ())),
                               preferred_element_type=out_dtype)


def _ka(ada_ref, w_ref, b_ref, mod_ref):
    a = _silu(ada_ref[...])
    mod_ref[...] = _dot_t(a, w_ref[...]) + b_ref[...]


def _kb(nbatch_blocks, x_ref, mod_ref, wq_ref, wk_ref, wv_ref, qb_ref,
        kb_ref, vb_ref, qr_ref, kr_ref, q_out, k_out, v_out):
    r = pl.program_id(0)
    b = r // nbatch_blocks
    x = x_ref[...]
    d = x.shape[1]
    ln = _ln(x)
    shift = mod_ref[pl.ds(b, 1), pl.ds(0, d)]
    scale = mod_ref[pl.ds(b, 1), pl.ds(d, d)]
    xm = (ln * (1.0 + scale) + shift).astype(BF16)
    q = _dot_t(xm, wq_ref[...]) + qb_ref[...]
    k = _dot_t(xm, wk_ref[...]) + kb_ref[...]
    v = _dot_t(xm, wv_ref[...]) + vb_ref[...]
    q = q * jax.lax.rsqrt(jnp.mean(q * q, axis=-1, keepdims=True) + 1e-5)
    q = q * qr_ref[...]
    k = k * jax.lax.rsqrt(jnp.mean(k * k, axis=-1, keepdims=True) + 1e-5)
    k = k * kr_ref[...]
    q_out[...] = (q * 0.125).astype(BF16)
    k_out[...] = k.astype(BF16)
    v_out[...] = v.astype(BF16)


def _kc(nheads, head_dim, q_ref, k_ref, v_ref, o_ref):
    outs = []
    for h in range(nheads):
        sl = slice(h * head_dim, (h + 1) * head_dim)
        qh = q_ref[:, sl]
        kh = k_ref[:, sl]
        vh = v_ref[:, sl]
        s = _dot_t(qh, kh)
        m = jnp.max(s, axis=-1, keepdims=True)
        p = jnp.exp(s - m)
        l = jnp.sum(p, axis=-1, keepdims=True)
        pb = (p / l).astype(BF16)
        outs.append(jax.lax.dot_general(
            pb, vh, (((1,), (0,)), ((), ())), preferred_element_type=F32))
    o_ref[...] = jnp.concatenate(outs, axis=1).astype(BF16)


def _kd(nbatch_blocks, nexp, attn_ref, x0_ref, mod_ref, wo_ref, ob_ref,
        gw_ref, x_out, n2_out, wts_out):
    r = pl.program_id(0)
    b = r // nbatch_blocks
    d = x0_ref.shape[1]
    a = _dot_t(attn_ref[...], wo_ref[...]) + ob_ref[...]
    gate_msa = mod_ref[pl.ds(b, 1), pl.ds(2 * d, d)]
    x = gate_msa * a + x0_ref[...]
    x_out[...] = x
    ln = _ln(x)
    shift_mlp = mod_ref[pl.ds(b, 1), pl.ds(3 * d, d)]
    scale_mlp = mod_ref[pl.ds(b, 1), pl.ds(4 * d, d)]
    n2 = ln * (1.0 + scale_mlp) + shift_mlp
    n2_out[...] = n2.astype(BF16)
    logits = _dot_t(n2, gw_ref[...])
    m = jnp.max(logits, axis=-1, keepdims=True)
    e = jnp.exp(logits - m)
    scores = e / jnp.sum(e, axis=-1, keepdims=True)
    cols = [scores[:, j:j + 1] for j in range(nexp)]
    wcols = []
    for i in range(nexp):
        cnt = jnp.zeros_like(cols[i])
        for j in range(nexp):
            if j == i:
                continue
            beats = (cols[j] > cols[i]) if j > i else (cols[j] >= cols[i])
            cnt = cnt + beats.astype(F32)
        wcols.append(jnp.where(cnt < 2.0, cols[i], 0.0))
    wts_out[...] = jnp.concatenate(wcols, axis=1)


def _ke(rb, nexp, n2_ref, w1_ref, w3_ref, w2_ref, wts_ref, out_ref):
    e = pl.program_id(0)
    h = pl.program_id(1)
    r = pl.program_id(2)
    rows = pl.ds(r * rb, rb)
    x = n2_ref[rows, :]
    w1 = w1_ref[0].astype(BF16)
    w3 = w3_ref[0].astype(BF16)
    g = _dot_t(x, w1)
    u = _dot_t(x, w3)
    wblk = wts_ref[rows, :]
    lane = jax.lax.broadcasted_iota(jnp.int32, wblk.shape, 1)
    we = jnp.sum(jnp.where(lane == e, wblk, 0.0), axis=1, keepdims=True)
    h1 = (_silu(g) * u * we).astype(BF16)
    w2 = w2_ref[0].astype(BF16)
    part = _dot_t(h1, w2)

    @pl.when(jnp.logical_and(e == 0, h == 0))
    def _init():
        out_ref[rows, :] = part

    @pl.when(jnp.logical_not(jnp.logical_and(e == 0, h == 0)))
    def _acc():
        out_ref[rows, :] = out_ref[rows, :] + part


def _kf(nbatch_blocks, n2_ref, moe_ref, x_ref, mod_ref, w1_ref, w3_ref,
        w2_ref, out_ref):
    r = pl.program_id(0)
    b = r // nbatch_blocks
    d = x_ref.shape[1]
    n2 = n2_ref[...]
    g = _dot_t(n2, w1_ref[...])
    u = _dot_t(n2, w3_ref[...])
    h1 = (_silu(g) * u).astype(BF16)
    sh = _dot_t(h1, w2_ref[...])
    y = moe_ref[...] + sh
    gate_mlp = mod_ref[pl.ds(b, 1), pl.ds(5 * d, d)]
    out_ref[...] = gate_mlp * y + x_ref[...]


def kernel(image_tokens, adaln_input, adaln_w, adaln_b, q_w, q_b, k_w, k_b,
           v_w, v_b, o_w, o_b, q_rms, k_rms, gate_w, experts_w1, experts_w2,
           experts_w3, shared_w1, shared_w2, shared_w3):
    bsz, s, d = image_tokens.shape
    nt = bsz * s
    nexp, exp_hid, _ = experts_w1.shape
    shared_hid = shared_w1.shape[0]
    nheads = d // 64
    head_dim = 64

    rb = 512 if s % 512 == 0 else s
    nbb = s // rb
    nrb = nt // rb
    hc = 256 if exp_hid % 256 == 0 else exp_hid
    nhc = exp_hid // hc

    x0 = image_tokens.reshape(nt, d)
    row2 = lambda a: a.reshape(1, -1)

    mod = pl.pallas_call(
        _ka,
        out_shape=jax.ShapeDtypeStruct((bsz, 6 * d), F32),
    )(adaln_input, adaln_w, row2(adaln_b))

    blk_row = pl.BlockSpec((rb, d), lambda r: (r, 0))
    full = lambda a: pl.BlockSpec(a.shape, lambda r: (0,) * a.ndim)
    wq, wk, wv = (w.astype(BF16) for w in (q_w, k_w, v_w))
    qb2, kb2, vb2 = row2(q_b), row2(k_b), row2(v_b)
    qr2, kr2 = row2(q_rms), row2(k_rms)
    q, k, v = pl.pallas_call(
        functools.partial(_kb, nbb),
        grid=(nrb,),
        in_specs=[blk_row, full(mod), full(wq), full(wk), full(wv),
                  full(qb2), full(kb2), full(vb2), full(qr2), full(kr2)],
        out_specs=(blk_row, blk_row, blk_row),
        out_shape=tuple(jax.ShapeDtypeStruct((nt, d), BF16) for _ in range(3)),
    )(x0, mod, wq, wk, wv, qb2, kb2, vb2, qr2, kr2)

    attn = pl.pallas_call(
        functools.partial(_kc, nheads, head_dim),
        grid=(bsz, nbb),
        in_specs=[
            pl.BlockSpec((rb, d), lambda b, qb: (b * nbb + qb, 0)),
            pl.BlockSpec((s, d), lambda b, qb: (b, 0)),
            pl.BlockSpec((s, d), lambda b, qb: (b, 0)),
        ],
        out_specs=pl.BlockSpec((rb, d), lambda b, qb: (b * nbb + qb, 0)),
        out_shape=jax.ShapeDtypeStruct((nt, d), BF16),
        compiler_params=pltpu.CompilerParams(
            dimension_semantics=("parallel", "arbitrary"),
        ),
    )(q, k, v)

    wo = o_w.astype(BF16)
    ob2, gw = row2(o_b), gate_w
    x, n2, wts = pl.pallas_call(
        functools.partial(_kd, nbb, nexp),
        grid=(nrb,),
        in_specs=[blk_row, blk_row, full(mod), full(wo), full(ob2), full(gw)],
        out_specs=(blk_row, blk_row, pl.BlockSpec((rb, nexp), lambda r: (r, 0))),
        out_shape=(jax.ShapeDtypeStruct((nt, d), F32),
                   jax.ShapeDtypeStruct((nt, d), BF16),
                   jax.ShapeDtypeStruct((nt, nexp), F32)),
    )(attn, x0, mod, wo, ob2, gw)

    moe = pl.pallas_call(
        functools.partial(_ke, rb, nexp),
        grid=(nexp, nhc, nrb),
        in_specs=[
            pl.BlockSpec((nt, d), lambda e, h, r: (0, 0)),
            pl.BlockSpec((1, hc, d), lambda e, h, r: (e, h, 0)),
            pl.BlockSpec((1, hc, d), lambda e, h, r: (e, h, 0)),
            pl.BlockSpec((1, d, hc), lambda e, h, r: (e, 0, h)),
            pl.BlockSpec((nt, nexp), lambda e, h, r: (0, 0)),
        ],
        out_specs=pl.BlockSpec((nt, d), lambda e, h, r: (0, 0)),
        out_shape=jax.ShapeDtypeStruct((nt, d), F32),
        compiler_params=pltpu.CompilerParams(
            dimension_semantics=("arbitrary", "arbitrary", "arbitrary"),
            vmem_limit_bytes=100 * 1024 * 1024,
        ),
    )(n2, experts_w1, experts_w3, experts_w2, wts)

    sw1, sw3, sw2 = (w.astype(BF16) for w in (shared_w1, shared_w3, shared_w2))
    out = pl.pallas_call(
        functools.partial(_kf, nbb),
        grid=(nrb,),
        in_specs=[blk_row, blk_row, blk_row, full(mod), full(sw1), full(sw3),
                  full(sw2)],
        out_specs=blk_row,
        out_shape=jax.ShapeDtypeStruct((nt, d), F32),
    )(n2, moe, x, mod, sw1, sw3, sw2)

    return out.reshape(bsz, s, d)


# speedup vs baseline: 1.4551x; 1.0622x over previous
import functools

import jax
import jax.numpy as jnp
from jax.experimental import pallas as pl
from jax.experimental.pallas import tpu as pltpu

F32 = jnp.float32
BF16 = jnp.bfloat16


def _silu(x):
    return x * jax.nn.sigmoid(x)


def _ln(x, eps=1e-6):
    mu = jnp.mean(x, axis=-1, keepdims=True)
    var = jnp.mean((x - mu) * (x - mu), axis=-1, keepdims=True)
    return (x - mu) * jax.lax.rsqrt(var + eps)


def _dot_t(a, b, out_dtype=F32):
    return jax.lax.dot_general(a, b, (((1,), (1,)), ((), ())),
                               preferred_element_type=out_dtype)


def _ka(ada_ref, w_ref, b_ref, mod_ref):
    a = _silu(ada_ref[...])
    mod_ref[...] = _dot_t(a, w_ref[...]) + b_ref[...]


def _kb(nbatch_blocks, x_ref, mod_ref, wq_ref, wk_ref, wv_ref, qb_ref,
        kb_ref, vb_ref, qr_ref, kr_ref, q_out, k_out, v_out):
    r = pl.program_id(0)
    b = r // nbatch_blocks
    x = x_ref[...]
    d = x.shape[1]
    ln = _ln(x)
    shift = mod_ref[pl.ds(b, 1), pl.ds(0, d)]
    scale = mod_ref[pl.ds(b, 1), pl.ds(d, d)]
    xm = (ln * (1.0 + scale) + shift).astype(BF16)
    q = _dot_t(xm, wq_ref[...]) + qb_ref[...]
    k = _dot_t(xm, wk_ref[...]) + kb_ref[...]
    v = _dot_t(xm, wv_ref[...]) + vb_ref[...]
    q = q * jax.lax.rsqrt(jnp.mean(q * q, axis=-1, keepdims=True) + 1e-5)
    q = q * qr_ref[...]
    k = k * jax.lax.rsqrt(jnp.mean(k * k, axis=-1, keepdims=True) + 1e-5)
    k = k * kr_ref[...]
    q_out[...] = (q * 0.125).astype(BF16)
    k_out[...] = k.astype(BF16)
    v_out[...] = v.astype(BF16)


def _kc(nheads, head_dim, q_ref, k_ref, v_ref, o_ref):
    outs = []
    for h in range(nheads):
        sl = slice(h * head_dim, (h + 1) * head_dim)
        qh = q_ref[:, sl]
        kh = k_ref[:, sl]
        vh = v_ref[:, sl]
        s = _dot_t(qh, kh)
        m = jnp.max(s, axis=-1, keepdims=True)
        p = jnp.exp(s - m)
        l = jnp.sum(p, axis=-1, keepdims=True)
        pb = (p / l).astype(BF16)
        outs.append(jax.lax.dot_general(
            pb, vh, (((1,), (0,)), ((), ())), preferred_element_type=F32))
    o_ref[...] = jnp.concatenate(outs, axis=1).astype(BF16)


def _kd(nbatch_blocks, nexp, attn_ref, x0_ref, mod_ref, wo_ref, ob_ref,
        gw_ref, x_out, n2_out, wts_out):
    r = pl.program_id(0)
    b = r // nbatch_blocks
    d = x0_ref.shape[1]
    a = _dot_t(attn_ref[...], wo_ref[...]) + ob_ref[...]
    gate_msa = mod_ref[pl.ds(b, 1), pl.ds(2 * d, d)]
    x = gate_msa * a + x0_ref[...]
    x_out[...] = x
    ln = _ln(x)
    shift_mlp = mod_ref[pl.ds(b, 1), pl.ds(3 * d, d)]
    scale_mlp = mod_ref[pl.ds(b, 1), pl.ds(4 * d, d)]
    n2 = ln * (1.0 + scale_mlp) + shift_mlp
    n2_out[...] = n2.astype(BF16)
    logits = _dot_t(n2, gw_ref[...])
    m = jnp.max(logits, axis=-1, keepdims=True)
    e = jnp.exp(logits - m)
    scores = e / jnp.sum(e, axis=-1, keepdims=True)
    cols = [scores[:, j:j + 1] for j in range(nexp)]
    wcols = []
    for i in range(nexp):
        cnt = jnp.zeros_like(cols[i])
        for j in range(nexp):
            if j == i:
                continue
            beats = (cols[j] > cols[i]) if j > i else (cols[j] >= cols[i])
            cnt = cnt + beats.astype(F32)
        wcols.append(jnp.where(cnt < 2.0, cols[i], 0.0))
    wts_out[...] = jnp.concatenate(wcols, axis=1)


def _ke(rb, nexp, n2_ref, w1_ref, w3_ref, w2_ref, wts_ref, out_ref):
    e = pl.program_id(0)
    h = pl.program_id(1)
    r = pl.program_id(2)
    rows = pl.ds(r * rb, rb)
    x = n2_ref[rows, :]
    g = _dot_t(x, w1_ref[0])
    u = _dot_t(x, w3_ref[0])
    wblk = wts_ref[rows, :]
    lane = jax.lax.broadcasted_iota(jnp.int32, wblk.shape, 1)
    we = jnp.sum(jnp.where(lane == e, wblk, 0.0), axis=1, keepdims=True)
    h1 = (_silu(g) * u * we).astype(BF16)
    part = _dot_t(h1, w2_ref[0])

    @pl.when(jnp.logical_and(e == 0, h == 0))
    def _init():
        out_ref[rows, :] = part

    @pl.when(jnp.logical_not(jnp.logical_and(e == 0, h == 0)))
    def _acc():
        out_ref[rows, :] = out_ref[rows, :] + part


def _kf(nbatch_blocks, n2_ref, moe_ref, x_ref, mod_ref, w1_ref, w3_ref,
        w2_ref, out_ref):
    r = pl.program_id(0)
    b = r // nbatch_blocks
    d = x_ref.shape[1]
    n2 = n2_ref[...]
    g = _dot_t(n2, w1_ref[...])
    u = _dot_t(n2, w3_ref[...])
    h1 = (_silu(g) * u).astype(BF16)
    sh = _dot_t(h1, w2_ref[...])
    y = moe_ref[...] + sh
    gate_mlp = mod_ref[pl.ds(b, 1), pl.ds(5 * d, d)]
    out_ref[...] = gate_mlp * y + x_ref[...]


def kernel(image_tokens, adaln_input, adaln_w, adaln_b, q_w, q_b, k_w, k_b,
           v_w, v_b, o_w, o_b, q_rms, k_rms, gate_w, experts_w1, experts_w2,
           experts_w3, shared_w1, shared_w2, shared_w3):
    bsz, s, d = image_tokens.shape
    nt = bsz * s
    nexp, exp_hid, _ = experts_w1.shape
    shared_hid = shared_w1.shape[0]
    nheads = d // 64
    head_dim = 64

    rb = 512 if s % 512 == 0 else s
    nbb = s // rb
    nrb = nt // rb
    hc = exp_hid // 2 if exp_hid % 256 == 0 else exp_hid
    nhc = exp_hid // hc

    x0 = image_tokens.reshape(nt, d)
    row2 = lambda a: a.reshape(1, -1)

    mod = pl.pallas_call(
        _ka,
        out_shape=jax.ShapeDtypeStruct((bsz, 6 * d), F32),
    )(adaln_input, adaln_w, row2(adaln_b))

    blk_row = pl.BlockSpec((rb, d), lambda r: (r, 0))
    full = lambda a: pl.BlockSpec(a.shape, lambda r: (0,) * a.ndim)
    wq, wk, wv = (w.astype(BF16) for w in (q_w, k_w, v_w))
    qb2, kb2, vb2 = row2(q_b), row2(k_b), row2(v_b)
    qr2, kr2 = row2(q_rms), row2(k_rms)
    q, k, v = pl.pallas_call(
        functools.partial(_kb, nbb),
        grid=(nrb,),
        in_specs=[blk_row, full(mod), full(wq), full(wk), full(wv),
                  full(qb2), full(kb2), full(vb2), full(qr2), full(kr2)],
        out_specs=(blk_row, blk_row, blk_row),
        out_shape=tuple(jax.ShapeDtypeStruct((nt, d), BF16) for _ in range(3)),
    )(x0, mod, wq, wk, wv, qb2, kb2, vb2, qr2, kr2)

    attn = pl.pallas_call(
        functools.partial(_kc, nheads, head_dim),
        grid=(bsz, nbb),
        in_specs=[
            pl.BlockSpec((rb, d), lambda b, qb: (b * nbb + qb, 0)),
            pl.BlockSpec((s, d), lambda b, qb: (b, 0)),
            pl.BlockSpec((s, d), lambda b, qb: (b, 0)),
        ],
        out_specs=pl.BlockSpec((rb, d), lambda b, qb: (b * nbb + qb, 0)),
        out_shape=jax.ShapeDtypeStruct((nt, d), BF16),
        compiler_params=pltpu.CompilerParams(
            dimension_semantics=("parallel", "arbitrary"),
        ),
    )(q, k, v)

    wo = o_w.astype(BF16)
    ob2, gw = row2(o_b), gate_w
    x, n2, wts = pl.pallas_call(
        functools.partial(_kd, nbb, nexp),
        grid=(nrb,),
        in_specs=[blk_row, blk_row, full(mod), full(wo), full(ob2), full(gw)],
        out_specs=(blk_row, blk_row, pl.BlockSpec((rb, nexp), lambda r: (r, 0))),
        out_shape=(jax.ShapeDtypeStruct((nt, d), F32),
                   jax.ShapeDtypeStruct((nt, d), BF16),
                   jax.ShapeDtypeStruct((nt, nexp), F32)),
    )(attn, x0, mod, wo, ob2, gw)

    ew1 = experts_w1.astype(BF16)
    ew2 = experts_w2.astype(BF16)
    ew3 = experts_w3.astype(BF16)
    moe = pl.pallas_call(
        functools.partial(_ke, rb, nexp),
        grid=(nexp, nhc, nrb),
        in_specs=[
            pl.BlockSpec((nt, d), lambda e, h, r: (0, 0)),
            pl.BlockSpec((1, hc, d), lambda e, h, r: (e, h, 0)),
            pl.BlockSpec((1, hc, d), lambda e, h, r: (e, h, 0)),
            pl.BlockSpec((1, d, hc), lambda e, h, r: (e, 0, h)),
            pl.BlockSpec((nt, nexp), lambda e, h, r: (0, 0)),
        ],
        out_specs=pl.BlockSpec((nt, d), lambda e, h, r: (0, 0)),
        out_shape=jax.ShapeDtypeStruct((nt, d), F32),
        compiler_params=pltpu.CompilerParams(
            dimension_semantics=("arbitrary", "arbitrary", "arbitrary"),
            vmem_limit_bytes=100 * 1024 * 1024,
        ),
    )(n2, ew1, ew3, ew2, wts)

    sw1, sw3, sw2 = (w.astype(BF16) for w in (shared_w1, shared_w3, shared_w2))
    out = pl.pallas_call(
        functools.partial(_kf, nbb),
        grid=(nrb,),
        in_specs=[blk_row, blk_row, blk_row, full(mod), full(sw1), full(sw3),
                  full(sw2)],
        out_specs=blk_row,
        out_shape=jax.ShapeDtypeStruct((nt, d), F32),
    )(n2, moe, x, mod, sw1, sw3, sw2)

    return out.reshape(bsz, s, d)


# speedup vs baseline: 1.6519x; 1.1352x over previous
import functools

import jax
import jax.numpy as jnp
from jax import lax
from jax.experimental import pallas as pl
from jax.experimental.pallas import tpu as pltpu
from jax.experimental.pallas import tpu_sc as plsc

F32 = jnp.float32
BF16 = jnp.bfloat16
I32 = jnp.int32


def _silu(x):
    return x * jax.nn.sigmoid(x)


def _ln(x, eps=1e-6):
    mu = jnp.mean(x, axis=-1, keepdims=True)
    var = jnp.mean((x - mu) * (x - mu), axis=-1, keepdims=True)
    return (x - mu) * jax.lax.rsqrt(var + eps)


def _dot_t(a, b, out_dtype=F32):
    return jax.lax.dot_general(a, b, (((1,), (1,)), ((), ())),
                               preferred_element_type=out_dtype)


def _ka(ada_ref, w_ref, b_ref, mod_ref):
    a = _silu(ada_ref[...])
    mod_ref[...] = _dot_t(a, w_ref[...]) + b_ref[...]


def _kb(nbatch_blocks, x_ref, mod_ref, wq_ref, wk_ref, wv_ref, qb_ref,
        kb_ref, vb_ref, qr_ref, kr_ref, q_out, k_out, v_out):
    r = pl.program_id(0)
    b = r // nbatch_blocks
    x = x_ref[...]
    d = x.shape[1]
    ln = _ln(x)
    shift = mod_ref[pl.ds(b, 1), pl.ds(0, d)]
    scale = mod_ref[pl.ds(b, 1), pl.ds(d, d)]
    xm = (ln * (1.0 + scale) + shift).astype(BF16)
    q = _dot_t(xm, wq_ref[...]) + qb_ref[...]
    k = _dot_t(xm, wk_ref[...]) + kb_ref[...]
    v = _dot_t(xm, wv_ref[...]) + vb_ref[...]
    q = q * jax.lax.rsqrt(jnp.mean(q * q, axis=-1, keepdims=True) + 1e-5)
    q = q * qr_ref[...]
    k = k * jax.lax.rsqrt(jnp.mean(k * k, axis=-1, keepdims=True) + 1e-5)
    k = k * kr_ref[...]
    q_out[...] = (q * 0.125).astype(BF16)
    k_out[...] = k.astype(BF16)
    v_out[...] = v.astype(BF16)


def _kc(nheads, head_dim, q_ref, k_ref, v_ref, o_ref):
    outs = []
    for h in range(nheads):
        sl = slice(h * head_dim, (h + 1) * head_dim)
        qh = q_ref[:, sl]
        kh = k_ref[:, sl]
        vh = v_ref[:, sl]
        s = _dot_t(qh, kh)
        m = jnp.max(s, axis=-1, keepdims=True)
        p = jnp.exp(s - m)
        l = jnp.sum(p, axis=-1, keepdims=True)
        pb = (p / l).astype(BF16)
        outs.append(jax.lax.dot_general(
            pb, vh, (((1,), (0,)), ((), ())), preferred_element_type=F32))
    o_ref[...] = jnp.concatenate(outs, axis=1).astype(BF16)


def _kd(nbatch_blocks, nexp, attn_ref, x0_ref, mod_ref, wo_ref, ob_ref,
        gw_ref, x_out, n2_out, n2f_out, wts_out):
    r = pl.program_id(0)
    b = r // nbatch_blocks
    d = x0_ref.shape[1]
    a = _dot_t(attn_ref[...], wo_ref[...]) + ob_ref[...]
    gate_msa = mod_ref[pl.ds(b, 1), pl.ds(2 * d, d)]
    x = gate_msa * a + x0_ref[...]
    x_out[...] = x
    ln = _ln(x)
    shift_mlp = mod_ref[pl.ds(b, 1), pl.ds(3 * d, d)]
    scale_mlp = mod_ref[pl.ds(b, 1), pl.ds(4 * d, d)]
    n2 = ln * (1.0 + scale_mlp) + shift_mlp
    n2_out[...] = n2.astype(BF16)
    n2f_out[...] = n2
    logits = _dot_t(n2, gw_ref[...])
    m = jnp.max(logits, axis=-1, keepdims=True)
    e = jnp.exp(logits - m)
    scores = e / jnp.sum(e, axis=-1, keepdims=True)
    cols = [scores[:, j:j + 1] for j in range(nexp)]
    wcols = []
    for i in range(nexp):
        cnt = jnp.zeros_like(cols[i])
        for j in range(nexp):
            if j == i:
                continue
            beats = (cols[j] > cols[i]) if j > i else (cols[j] >= cols[i])
            cnt = cnt + beats.astype(F32)
        wcols.append(jnp.where(cnt < 2.0, cols[i], 0.0))
    wts_out[...] = jnp.concatenate(wcols, axis=1)


def _kr(nrb, blk_rows, nbmax, wts_ref, s1_ref, s2_ref, w1_ref, w2_ref,
        blk_ref, carry_ref, poff_ref):
    p = pl.program_id(0)
    r = pl.program_id(1)
    rb, nexp = wts_ref.shape

    @pl.when(r == 0)
    def _reset():
        carry_ref[...] = jnp.zeros_like(carry_ref)

    wts = wts_ref[...]
    a = (wts > 0.0)
    abf = a.astype(BF16)
    li = lax.broadcasted_iota(I32, (rb, rb), 0)
    lj = lax.broadcasted_iota(I32, (rb, rb), 1)
    ltri = (lj < li).astype(BF16)
    pref = jax.lax.dot_general(ltri, abf, (((1,), (0,)), ((), ())),
                               preferred_element_type=F32)
    carry = carry_ref[...]
    rank = pref + carry
    carry_ref[...] = carry + jnp.sum(a.astype(F32), axis=0, keepdims=True)

    @pl.when(jnp.logical_and(p == 0, r == nrb - 1))
    def _mk_offsets():
        c = carry_ref[...]
        nblk = jnp.floor((c + (blk_rows - 1.0)) * (1.0 / blk_rows))
        cums = []
        acc = nblk[:, 0:1]
        cums.append(acc)
        for e in range(1, nexp):
            acc = acc + nblk[:, e:e + 1]
            cums.append(acc)
        poff = jnp.concatenate(
            [jnp.zeros_like(cums[0])] + cums[:-1], axis=1) * blk_rows
        poff_ref[...] = poff
        gi = lax.broadcasted_iota(I32, (1, nbmax), 1).astype(F32)
        eid = jnp.zeros((1, nbmax), F32)
        for e in range(nexp - 1):
            eid = eid + (gi >= cums[e]).astype(F32)
        blk_ref[...] = eid.astype(I32)

    @pl.when(p == 1)
    def _emit_slots():
        poff = poff_ref[...]
        slotm = poff + rank
        big = jnp.float32(1e9)
        s1v = jnp.min(jnp.where(a, slotm, big), axis=1, keepdims=True)
        s2v = jnp.max(jnp.where(a, slotm, -big), axis=1, keepdims=True)
        w1v = jnp.sum(jnp.where(slotm == s1v, wts, 0.0), axis=1,
                      keepdims=True)
        w2v = jnp.sum(jnp.where(slotm == s2v, wts, 0.0), axis=1,
                      keepdims=True)
        s1_ref[...] = s1v.astype(I32)
        s2_ref[...] = s2v.astype(I32)
        w1_ref[...] = w1v
        w2_ref[...] = w2v


def _sc_scatter_rows(n2f, s1, s2, ns):
    nt, d = n2f.shape
    nw = 32
    per_w = nt // nw
    chunk = 32
    nch = per_w // chunk
    mesh = plsc.VectorSubcoreMesh(core_axis_name="c", subcore_axis_name="s")

    def body(n2_hbm, s1_hbm, s2_hbm, xs_hbm, rows_v, idx_v, sem):
        wid = lax.axis_index("s") * 2 + lax.axis_index("c")
        base = wid * per_w

        @pl.loop(0, nch)
        def _(i):
            b = base + i * chunk
            pltpu.sync_copy(n2_hbm.at[pl.ds(b, chunk)], rows_v)
            pltpu.sync_copy(s1_hbm.at[pl.ds(b, chunk)], idx_v)
            pltpu.async_copy(rows_v, xs_hbm.at[idx_v], sem).wait()
            pltpu.sync_copy(s2_hbm.at[pl.ds(b, chunk)], idx_v)
            pltpu.async_copy(rows_v, xs_hbm.at[idx_v], sem).wait()

    return pl.kernel(
        body,
        out_type=jax.ShapeDtypeStruct((ns, d), F32),
        mesh=mesh,
        scratch_types=[pltpu.VMEM((chunk, d), F32),
                       pltpu.VMEM((chunk,), I32),
                       pltpu.SemaphoreType.DMA],
    )(n2f, s1, s2)


def _sc_gather_rows(ys, s1, s2):
    ns, d = ys.shape
    nt = s1.shape[0]
    nw = 32
    per_w = nt // nw
    chunk = 32
    nch = per_w // chunk
    mesh = plsc.VectorSubcoreMesh(core_axis_name="c", subcore_axis_name="s")

    def body(ys_hbm, s1_hbm, s2_hbm, g1_hbm, g2_hbm, rows_v, idx_v, sem):
        wid = lax.axis_index("s") * 2 + lax.axis_index("c")
        base = wid * per_w

        @pl.loop(0, nch)
        def _(i):
            b = base + i * chunk
            pltpu.sync_copy(s1_hbm.at[pl.ds(b, chunk)], idx_v)
            pltpu.async_copy(ys_hbm.at[idx_v], rows_v, sem).wait()
            pltpu.sync_copy(rows_v, g1_hbm.at[pl.ds(b, chunk)])
            pltpu.sync_copy(s2_hbm.at[pl.ds(b, chunk)], idx_v)
            pltpu.async_copy(ys_hbm.at[idx_v], rows_v, sem).wait()
            pltpu.sync_copy(rows_v, g2_hbm.at[pl.ds(b, chunk)])

    return pl.kernel(
        body,
        out_type=(jax.ShapeDtypeStruct((nt, d), F32),
                  jax.ShapeDtypeStruct((nt, d), F32)),
        mesh=mesh,
        scratch_types=[pltpu.VMEM((chunk, d), F32),
                       pltpu.VMEM((chunk,), I32),
                       pltpu.SemaphoreType.DMA],
    )(ys, s1, s2)


def _ks(eid_ref, xs_ref, w1_ref, w3_ref, w2_ref, ys_ref):
    x = xs_ref[...].astype(BF16)
    g = _dot_t(x, w1_ref[0])
    u = _dot_t(x, w3_ref[0])
    h1 = (_silu(g) * u).astype(BF16)
    ys_ref[...] = _dot_t(h1, w2_ref[0])


def _kf(nbatch_blocks, n2_ref, g1_ref, g2_ref, wt1_ref, wt2_ref, x_ref,
        mod_ref, w1_ref, w3_ref, w2_ref, out_ref):
    r = pl.program_id(0)
    b = r // nbatch_blocks
    d = x_ref.shape[1]
    n2 = n2_ref[...]
    g = _dot_t(n2, w1_ref[...])
    u = _dot_t(n2, w3_ref[...])
    h1 = (_silu(g) * u).astype(BF16)
    sh = _dot_t(h1, w2_ref[...])
    y = wt1_ref[...] * g1_ref[...] + wt2_ref[...] * g2_ref[...] + sh
    gate_mlp = mod_ref[pl.ds(b, 1), pl.ds(5 * d, d)]
    out_ref[...] = gate_mlp * y + x_ref[...]


def kernel(image_tokens, adaln_input, adaln_w, adaln_b, q_w, q_b, k_w, k_b,
           v_w, v_b, o_w, o_b, q_rms, k_rms, gate_w, experts_w1, experts_w2,
           experts_w3, shared_w1, shared_w2, shared_w3):
    bsz, s, d = image_tokens.shape
    nt = bsz * s
    nexp, exp_hid, _ = experts_w1.shape
    shared_hid = shared_w1.shape[0]
    nheads = d // 64
    head_dim = 64

    rb = 512 if s % 512 == 0 else s
    nbb = s // rb
    nrb = nt // rb

    x0 = image_tokens.reshape(nt, d)
    row2 = lambda a: a.reshape(1, -1)

    mod = pl.pallas_call(
        _ka,
        out_shape=jax.ShapeDtypeStruct((bsz, 6 * d), F32),
    )(adaln_input, adaln_w, row2(adaln_b))

    blk_row = pl.BlockSpec((rb, d), lambda r: (r, 0))
    full = lambda a: pl.BlockSpec(a.shape, lambda r: (0,) * a.ndim)
    wq, wk, wv = (w.astype(BF16) for w in (q_w, k_w, v_w))
    qb2, kb2, vb2 = row2(q_b), row2(k_b), row2(v_b)
    qr2, kr2 = row2(q_rms), row2(k_rms)
    q, k, v = pl.pallas_call(
        functools.partial(_kb, nbb),
        grid=(nrb,),
        in_specs=[blk_row, full(mod), full(wq), full(wk), full(wv),
                  full(qb2), full(kb2), full(vb2), full(qr2), full(kr2)],
        out_specs=(blk_row, blk_row, blk_row),
        out_shape=tuple(jax.ShapeDtypeStruct((nt, d), BF16) for _ in range(3)),
    )(x0, mod, wq, wk, wv, qb2, kb2, vb2, qr2, kr2)

    attn = pl.pallas_call(
        functools.partial(_kc, nheads, head_dim),
        grid=(bsz, nbb),
        in_specs=[
            pl.BlockSpec((rb, d), lambda b, qb: (b * nbb + qb, 0)),
            pl.BlockSpec((s, d), lambda b, qb: (b, 0)),
            pl.BlockSpec((s, d), lambda b, qb: (b, 0)),
        ],
        out_specs=pl.BlockSpec((rb, d), lambda b, qb: (b * nbb + qb, 0)),
        out_shape=jax.ShapeDtypeStruct((nt, d), BF16),
        compiler_params=pltpu.CompilerParams(
            dimension_semantics=("parallel", "arbitrary"),
        ),
    )(q, k, v)

    wo = o_w.astype(BF16)
    ob2, gw = row2(o_b), gate_w
    x, n2, n2f, wts = pl.pallas_call(
        functools.partial(_kd, nbb, nexp),
        grid=(nrb,),
        in_specs=[blk_row, blk_row, full(mod), full(wo), full(ob2), full(gw)],
        out_specs=(blk_row, blk_row, blk_row,
                   pl.BlockSpec((rb, nexp), lambda r: (r, 0))),
        out_shape=(jax.ShapeDtypeStruct((nt, d), F32),
                   jax.ShapeDtypeStruct((nt, d), BF16),
                   jax.ShapeDtypeStruct((nt, d), F32),
                   jax.ShapeDtypeStruct((nt, nexp), F32)),
    )(attn, x0, mod, wo, ob2, gw)

    br = 256
    nbmax = (2 * nt) // br + nexp - 1
    ns = nbmax * br
    s1, s2, wt1, wt2, blk_eid = pl.pallas_call(
        functools.partial(_kr, nrb, float(br), nbmax),
        grid=(2, nrb),
        in_specs=[pl.BlockSpec((rb, nexp), lambda p, r: (r, 0))],
        out_specs=(pl.BlockSpec((rb, 1), lambda p, r: (r, 0)),
                   pl.BlockSpec((rb, 1), lambda p, r: (r, 0)),
                   pl.BlockSpec((rb, 1), lambda p, r: (r, 0)),
                   pl.BlockSpec((rb, 1), lambda p, r: (r, 0)),
                   pl.BlockSpec((1, nbmax), lambda p, r: (0, 0))),
        out_shape=(jax.ShapeDtypeStruct((nt, 1), I32),
                   jax.ShapeDtypeStruct((nt, 1), I32),
                   jax.ShapeDtypeStruct((nt, 1), F32),
                   jax.ShapeDtypeStruct((nt, 1), F32),
                   jax.ShapeDtypeStruct((1, nbmax), I32)),
        scratch_shapes=[pltpu.VMEM((1, nexp), F32),
                        pltpu.VMEM((1, nexp), F32)],
        compiler_params=pltpu.CompilerParams(
            dimension_semantics=("arbitrary", "arbitrary")),
    )(wts)
    s1f = s1.reshape(nt)
    s2f = s2.reshape(nt)

    xs = _sc_scatter_rows(n2f, s1f, s2f, ns)

    ew1 = experts_w1.astype(BF16)
    ew2 = experts_w2.astype(BF16)
    ew3 = experts_w3.astype(BF16)
    ys = pl.pallas_call(
        _ks,
        grid_spec=pltpu.PrefetchScalarGridSpec(
            num_scalar_prefetch=1,
            grid=(nbmax,),
            in_specs=[
                pl.BlockSpec((br, d), lambda g, eid: (g, 0)),
                pl.BlockSpec((1, exp_hid, d), lambda g, eid: (eid[0, g], 0, 0)),
                pl.BlockSpec((1, exp_hid, d), lambda g, eid: (eid[0, g], 0, 0)),
                pl.BlockSpec((1, d, exp_hid), lambda g, eid: (eid[0, g], 0, 0)),
            ],
            out_specs=pl.BlockSpec((br, d), lambda g, eid: (g, 0)),
        ),
        out_shape=jax.ShapeDtypeStruct((ns, d), F32),
        compiler_params=pltpu.CompilerParams(
            dimension_semantics=("arbitrary",),
            vmem_limit_bytes=100 * 1024 * 1024,
        ),
    )(blk_eid, xs, ew1, ew3, ew2)

    g1, g2 = _sc_gather_rows(ys, s1f, s2f)

    sw1, sw3, sw2 = (w.astype(BF16) for w in (shared_w1, shared_w3, shared_w2))
    wcol = pl.BlockSpec((rb, 1), lambda r: (r, 0))
    out = pl.pallas_call(
        functools.partial(_kf, nbb),
        grid=(nrb,),
        in_specs=[blk_row, blk_row, blk_row, wcol, wcol, blk_row, full(mod),
                  full(sw1), full(sw3), full(sw2)],
        out_specs=blk_row,
        out_shape=jax.ShapeDtypeStruct((nt, d), F32),
    )(n2, g1, g2, wt1, wt2, x, mod, sw1, sw3, sw2)

    return out.reshape(bsz, s, d)
